# baseline (speedup 1.0000x reference)
import functools

import jax
import jax.numpy as jnp
from jax import lax
from jax.experimental import pallas as pl
from jax.experimental.pallas import tpu as pltpu
from jax.experimental.pallas import tpu_sc as plsc

H = 32
NE1 = 1537
DSTRIDE = 1544
NSPATIAL = 512
SP_BASE = 5 * DSTRIDE
TBL = SP_BASE + NSPATIAL
B, N = 32, 64
M = B * N * N

NC, NS = 2, 16
NW = NC * NS
M_PER_W = M // NW
CHUNK = 128
NCHUNK = M_PER_W // CHUNK
E_LOOK = CHUNK * 15
NSTREAM = 15
IE_ROWS = M * 15 // 128
IS_ROWS = M // 128


def _table_body(e_ref, wm_ref, spw_ref, tbl_ref):
    for d in range(5):
        ft = jnp.dot(e_ref[...], wm_ref[d], preferred_element_type=jnp.float32)
        tbl_ref[pl.ds(d * DSTRIDE, DSTRIDE), :] = ft.astype(jnp.bfloat16)
    tbl_ref[pl.ds(SP_BASE, NSPATIAL), :] = spw_ref[...].astype(jnp.bfloat16)


def _build_table(e_pad, wm, spw):
    return pl.pallas_call(
        _table_body,
        out_shape=jax.ShapeDtypeStruct((TBL, H), jnp.bfloat16),
    )(e_pad, wm, spw)


def _prep_body(ef_ref, sp_ref, ie_ref, is_ref):
    er = ef_ref.shape[0]
    q = (lax.broadcasted_iota(jnp.int32, (er, 128), 0) * 128
         + lax.broadcasted_iota(jnp.int32, (er, 128), 1))
    p = q % 15
    ie_ref[...] = ef_ref[...] + (p // 3) * DSTRIDE
    is_ref[...] = sp_ref[...] + SP_BASE


def _build_indices(ef, sp2):
    n_chunks = 16
    erows = ef.shape[0] // n_chunks
    srows = sp2.shape[0] // n_chunks
    return pl.pallas_call(
        _prep_body,
        grid=(n_chunks,),
        in_specs=[
            pl.BlockSpec((erows, 128), lambda c: (c, 0)),
            pl.BlockSpec((srows, 128), lambda c: (c, 0)),
        ],
        out_specs=[
            pl.BlockSpec((erows, 128), lambda c: (c, 0)),
            pl.BlockSpec((srows, 128), lambda c: (c, 0)),
        ],
        out_shape=[
            jax.ShapeDtypeStruct(ef.shape, jnp.int32),
            jax.ShapeDtypeStruct(sp2.shape, jnp.int32),
        ],
    )(ef, sp2)


def _sc_body(tbl_hbm, ie_hbm, is_hbm, eo_e_hbm, eo_s_hbm,
             ie_v, is_v, rows_e, rows_s, out_v, semA, semB):
    wid = lax.axis_index("s") * NC + lax.axis_index("c")
    sems = (semA, semB)

    def stage_and_fire(g, buf):
        er0 = wid * (NCHUNK * NSTREAM) + g * NSTREAM
        sr = wid * NCHUNK + g
        pltpu.sync_copy(ie_hbm.at[pl.ds(er0, NSTREAM), :], ie_v.at[buf])
        pltpu.sync_copy(is_hbm.at[pl.ds(sr, 1), :], is_v.at[buf])
        for r in range(NSTREAM):
            pltpu.async_copy(
                tbl_hbm.at[ie_v.at[buf, r]],
                rows_e.at[buf, pl.ds(r * 128, 128)],
                sems[buf],
            )
        pltpu.async_copy(
            tbl_hbm.at[is_v.at[buf, 0]], rows_s.at[buf], sems[buf])

    def drain(buf):
        pltpu.make_async_copy(
            tbl_hbm.at[pl.ds(0, E_LOOK)], rows_e.at[buf], sems[buf]).wait()
        pltpu.make_async_copy(
            tbl_hbm.at[pl.ds(0, CHUNK)], rows_s.at[buf], sems[buf]).wait()

    def compute(g, buf):
        def out_body(o, _):
            base = o * 15
            acc = rows_e[buf, base, :]
            for k in range(1, 15):
                acc = acc + rows_e[buf, base + k, :]
            out_v[o, :] = acc
            return 0

        lax.fori_loop(0, CHUNK, out_body, 0)
        m0 = wid * M_PER_W + g * CHUNK
        pltpu.sync_copy(out_v, eo_e_hbm.at[pl.ds(m0, CHUNK)])
        pltpu.sync_copy(rows_s.at[buf], eo_s_hbm.at[pl.ds(m0, CHUNK)])

    stage_and_fire(0, 0)

    def body2(i, _):
        g0 = 2 * i
        stage_and_fire(g0 + 1, 1)
        drain(0)
        compute(g0, 0)

        @pl.when(g0 + 2 < NCHUNK)
        def _():
            stage_and_fire(g0 + 2, 0)

        drain(1)
        compute(g0 + 1, 1)
        return 0

    lax.fori_loop(0, NCHUNK // 2, body2, 0)


def _sc_gather(table, idx_e, idx_s):
    mesh = plsc.VectorSubcoreMesh(core_axis_name="c", subcore_axis_name="s")
    run = pl.kernel(
        _sc_body,
        out_type=(
            jax.ShapeDtypeStruct((M, H), jnp.bfloat16),
            jax.ShapeDtypeStruct((M, H), jnp.bfloat16),
        ),
        mesh=mesh,
        compiler_params=pltpu.CompilerParams(use_tc_tiling_on_sc=False),
        scratch_types=[
            pltpu.VMEM((2, NSTREAM, 128), jnp.int32),
            pltpu.VMEM((2, 1, 128), jnp.int32),
            pltpu.VMEM((2, E_LOOK, H), jnp.bfloat16),
            pltpu.VMEM((2, CHUNK, H), jnp.bfloat16),
            pltpu.VMEM((CHUNK, H), jnp.bfloat16),
            pltpu.SemaphoreType.DMA,
            pltpu.SemaphoreType.DMA,
        ],
    )
    return run(table, idx_e, idx_s)


def _assemble_body(ab_ref, ee_ref, es_ref, sp_ref, gt_ref, out_ref):
    ee = ee_ref[...].astype(jnp.float32)
    es = es_ref[...].astype(jnp.float32)
    et = ee.T.reshape(H, N, N)
    st = es.T.reshape(H, N, N)
    sp = sp_ref[0]
    sp_ = jnp.where(sp == 0, 1, sp)
    sp_ = jnp.where(sp_ > 1, sp_ - 1, sp_)
    sp_ = jnp.clip(sp_, 0, 5)
    scl = 1.0 / (3.0 * sp_.astype(jnp.float32))
    core = et * scl[None, :, :] + st
    zr = jnp.zeros((H, 1, N), jnp.float32)
    tmp = jnp.concatenate([zr, core], axis=1)
    zc = jnp.zeros((H, N + 1, 1), jnp.float32)
    padded = jnp.concatenate([zc, tmp], axis=2)
    ii = lax.broadcasted_iota(jnp.int32, (H, N + 1, N + 1), 1)
    jj = lax.broadcasted_iota(jnp.int32, (H, N + 1, N + 1), 2)
    t = gt_ref[0, :].reshape(H, 1, 1)
    tfield = jnp.where((ii == 0) | (jj == 0), t, 0.0)
    out_ref[0] = 2.0 * ab_ref[0][None, :, :] + padded + tfield


def _assemble(attn_bias, eo_e, eo_s, spatial_pos, graph_token_w):
    return pl.pallas_call(
        _assemble_body,
        grid=(B,),
        in_specs=[
            pl.BlockSpec((1, N + 1, N + 1), lambda b: (b, 0, 0)),
            pl.BlockSpec((N * N, H), lambda b: (b, 0)),
            pl.BlockSpec((N * N, H), lambda b: (b, 0)),
            pl.BlockSpec((1, N, N), lambda b: (b, 0, 0)),
            pl.BlockSpec((1, H), lambda b: (0, 0)),
        ],
        out_specs=pl.BlockSpec((1, H, N + 1, N + 1), lambda b: (b, 0, 0, 0)),
        out_shape=jax.ShapeDtypeStruct((B, H, N + 1, N + 1), jnp.float32),
    )(attn_bias, eo_e, eo_s, spatial_pos, graph_token_w)


def kernel(attn_bias, spatial_pos, x, edge_input, attn_edge_type,
           edge_encoder_w, spatial_pos_encoder_w, graph_token_w,
           edge_dis_encoder_w):
    spatial_pos = spatial_pos.astype(jnp.int32)
    edge_input = edge_input.astype(jnp.int32)

    e_pad = jnp.pad(edge_encoder_w, ((0, DSTRIDE - NE1), (0, 0)))
    wm = edge_dis_encoder_w.reshape(128, H, H)[:5]
    table = _build_table(e_pad, wm, spatial_pos_encoder_w)

    ef = edge_input.reshape(IE_ROWS, 128)
    sp2 = spatial_pos.reshape(IS_ROWS, 128)
    idx_e, idx_s = _build_indices(ef, sp2)

    eo_e, eo_s = _sc_gather(table, idx_e, idx_s)
    return _assemble(attn_bias, eo_e, eo_s, spatial_pos, graph_token_w)


# speedup vs baseline: 2.9603x; 2.9603x over previous
import functools

import jax
import jax.numpy as jnp
from jax import lax
from jax.experimental import pallas as pl
from jax.experimental.pallas import tpu as pltpu
from jax.experimental.pallas import tpu_sc as plsc

H = 32
NE1 = 1537
DSTRIDE = 1544
SSTRIDE = 5 * DSTRIDE
NSPATIAL = 512
SP_BASE = 5 * SSTRIDE
TBL = SP_BASE + NSPATIAL
B, N = 32, 64
M = B * N * N

NC, NS = 2, 16
NW = NC * NS
M_PER_W = M // NW
CHUNK = 128
NCHUNK = M_PER_W // CHUNK
LOOKUPS = CHUNK * 16
NSTREAM = 16
IDX_ROWS = M * 16 // 128


def _table_body(e_ref, wm_ref, spw_ref, tbl_ref):
    for d in range(5):
        ft = jnp.dot(e_ref[...], wm_ref[d], preferred_element_type=jnp.float32)
        for s in range(5):
            tbl_ref[pl.ds(s * SSTRIDE + d * DSTRIDE, DSTRIDE), :] = (
                ft * (1.0 / (3.0 * (s + 1)))).astype(jnp.bfloat16)
    tbl_ref[pl.ds(SP_BASE, NSPATIAL), :] = spw_ref[...].astype(jnp.bfloat16)


def _build_table(e_pad, wm, spw):
    return pl.pallas_call(
        _table_body,
        out_shape=jax.ShapeDtypeStruct((TBL, H), jnp.bfloat16),
    )(e_pad, wm, spw)


def _idx_body(cat_ref, spx_ref, idx_ref):
    p = lax.broadcasted_iota(jnp.int32, (1, 128), 1) % 16
    sp = spx_ref[...]
    sp_ = jnp.where(sp == 0, 1, sp)
    sp_ = jnp.where(sp_ > 1, sp_ - 1, sp_)
    sp_ = jnp.clip(sp_, 0, 5)
    offs = jnp.where(p == 15, SP_BASE,
                     (p // 3) * DSTRIDE + (sp_ - 1) * SSTRIDE)
    idx_ref[...] = cat_ref[...] + offs


def _build_idx(cat2, spx2):
    n_chunks = 16
    rows = cat2.shape[0] // n_chunks
    return pl.pallas_call(
        _idx_body,
        grid=(n_chunks,),
        in_specs=[
            pl.BlockSpec((rows, 128), lambda c: (c, 0)),
            pl.BlockSpec((rows, 128), lambda c: (c, 0)),
        ],
        out_specs=pl.BlockSpec((rows, 128), lambda c: (c, 0)),
        out_shape=jax.ShapeDtypeStruct(cat2.shape, jnp.int32),
    )(cat2, spx2)


def _sc_body(tbl_hbm, idx_hbm, out_hbm, idx_v, rows_v, out_v, semA, semB):
    wid = lax.axis_index("s") * NC + lax.axis_index("c")
    sems = (semA, semB)

    def stage_and_fire(g, buf):
        r0 = wid * (NCHUNK * NSTREAM) + g * NSTREAM
        pltpu.sync_copy(idx_hbm.at[pl.ds(r0, NSTREAM), :], idx_v.at[buf])
        for r in range(NSTREAM):
            pltpu.async_copy(
                tbl_hbm.at[idx_v.at[buf, r]],
                rows_v.at[buf, pl.ds(r * 128, 128)],
                sems[buf],
            )

    def drain(buf):
        pltpu.make_async_copy(
            tbl_hbm.at[pl.ds(0, LOOKUPS)], rows_v.at[buf], sems[buf]).wait()

    def compute(g, buf):
        def out_body(o, _):
            base = o * 16
            acc = rows_v[buf, base, :]
            for k in range(1, 16):
                acc = acc + rows_v[buf, base + k, :]
            out_v[o, :] = acc
            return 0

        lax.fori_loop(0, CHUNK, out_body, 0)
        m0 = wid * M_PER_W + g * CHUNK
        pltpu.sync_copy(out_v, out_hbm.at[pl.ds(m0, CHUNK)])

    stage_and_fire(0, 0)

    def body2(i, _):
        g0 = 2 * i
        stage_and_fire(g0 + 1, 1)
        drain(0)
        compute(g0, 0)

        @pl.when(g0 + 2 < NCHUNK)
        def _():
            stage_and_fire(g0 + 2, 0)

        drain(1)
        compute(g0 + 1, 1)
        return 0

    lax.fori_loop(0, NCHUNK // 2, body2, 0)


def _sc_gather(table, idx2):
    mesh = plsc.VectorSubcoreMesh(core_axis_name="c", subcore_axis_name="s")
    run = pl.kernel(
        _sc_body,
        out_type=jax.ShapeDtypeStruct((M, H), jnp.bfloat16),
        mesh=mesh,
        compiler_params=pltpu.CompilerParams(use_tc_tiling_on_sc=False),
        scratch_types=[
            pltpu.VMEM((2, NSTREAM, 128), jnp.int32),
            pltpu.VMEM((2, LOOKUPS, H), jnp.bfloat16),
            pltpu.VMEM((CHUNK, H), jnp.bfloat16),
            pltpu.SemaphoreType.DMA,
            pltpu.SemaphoreType.DMA,
        ],
    )
    return run(table, idx2)


def _assemble_body(ab_ref, e_ref, gt_ref, out_ref):
    e = e_ref[...].astype(jnp.float32)
    et = e.T.reshape(H, N, N)
    zr = jnp.zeros((H, 1, N), jnp.float32)
    tmp = jnp.concatenate([zr, et], axis=1)
    zc = jnp.zeros((H, N + 1, 1), jnp.float32)
    padded = jnp.concatenate([zc, tmp], axis=2)
    ii = lax.broadcasted_iota(jnp.int32, (H, N + 1, N + 1), 1)
    jj = lax.broadcasted_iota(jnp.int32, (H, N + 1, N + 1), 2)
    t = gt_ref[0, :].reshape(H, 1, 1)
    tfield = jnp.where((ii == 0) | (jj == 0), t, 0.0)
    out_ref[0] = 2.0 * ab_ref[0][None, :, :] + padded + tfield


def _assemble(attn_bias, eout, graph_token_w):
    return pl.pallas_call(
        _assemble_body,
        grid=(B,),
        in_specs=[
            pl.BlockSpec((1, N + 1, N + 1), lambda b: (b, 0, 0)),
            pl.BlockSpec((N * N, H), lambda b: (b, 0)),
            pl.BlockSpec((1, H), lambda b: (0, 0)),
        ],
        out_specs=pl.BlockSpec((1, H, N + 1, N + 1), lambda b: (b, 0, 0, 0)),
        out_shape=jax.ShapeDtypeStruct((B, H, N + 1, N + 1), jnp.float32),
    )(attn_bias, eout, graph_token_w)


def kernel(attn_bias, spatial_pos, x, edge_input, attn_edge_type,
           edge_encoder_w, spatial_pos_encoder_w, graph_token_w,
           edge_dis_encoder_w):
    spatial_pos = spatial_pos.astype(jnp.int32)
    edge_input = edge_input.astype(jnp.int32)

    e_pad = jnp.pad(edge_encoder_w, ((0, DSTRIDE - NE1), (0, 0)))
    wm = edge_dis_encoder_w.reshape(128, H, H)[:5]
    table = _build_table(e_pad, wm, spatial_pos_encoder_w)

    cat = jnp.concatenate(
        [edge_input.reshape(M, 15), spatial_pos.reshape(M, 1)], axis=1)
    spx = jnp.broadcast_to(spatial_pos.reshape(M, 1), (M, 16))
    idx2 = _build_idx(cat.reshape(IDX_ROWS, 128), spx.reshape(IDX_ROWS, 128))

    eout = _sc_gather(table, idx2)
    return _assemble(attn_bias, eout, graph_token_w)


# speedup vs baseline: 3.0928x; 1.0448x over previous
import functools

import jax
import jax.numpy as jnp
from jax import lax
from jax.experimental import pallas as pl
from jax.experimental.pallas import tpu as pltpu
from jax.experimental.pallas import tpu_sc as plsc

H = 32
NE1 = 1537
DSTRIDE = 1544
SSTRIDE = 5 * DSTRIDE
NSPATIAL = 512
SP_BASE = 5 * SSTRIDE
TBL = SP_BASE + NSPATIAL
B, N = 32, 64
M = B * N * N

NC, NS = 2, 16
NW = NC * NS
M_PER_W = M // NW
CHUNK = 128
NCHUNK = M_PER_W // CHUNK
LOOKUPS = CHUNK * 16
NSTREAM = 16
IDX_ROWS = M * 16 // 128


def _table_body(e_ref, wm_ref, spw_ref, tbl_ref):
    for d in range(5):
        ft = jnp.dot(e_ref[...], wm_ref[d], preferred_element_type=jnp.float32)
        for s in range(5):
            tbl_ref[pl.ds(s * SSTRIDE + d * DSTRIDE, DSTRIDE), :] = (
                ft * (1.0 / (3.0 * (s + 1)))).astype(jnp.bfloat16)
    tbl_ref[pl.ds(SP_BASE, NSPATIAL), :] = spw_ref[...].astype(jnp.bfloat16)


def _build_table(e_pad, wm, spw):
    return pl.pallas_call(
        _table_body,
        out_shape=jax.ShapeDtypeStruct((TBL, H), jnp.bfloat16),
    )(e_pad, wm, spw)


def _idx_body(cat_ref, spx_ref, idx_ref):
    p = lax.broadcasted_iota(jnp.int32, (1, 128), 1) % 16
    sp = spx_ref[...]
    sp_ = jnp.where(sp == 0, 1, sp)
    sp_ = jnp.where(sp_ > 1, sp_ - 1, sp_)
    sp_ = jnp.clip(sp_, 0, 5)
    offs = jnp.where(p == 15, SP_BASE,
                     (p // 3) * DSTRIDE + (sp_ - 1) * SSTRIDE)
    idx_ref[...] = cat_ref[...] + offs


def _build_idx(cat2, spx2):
    n_chunks = 16
    rows = cat2.shape[0] // n_chunks
    return pl.pallas_call(
        _idx_body,
        grid=(n_chunks,),
        in_specs=[
            pl.BlockSpec((rows, 128), lambda c: (c, 0)),
            pl.BlockSpec((rows, 128), lambda c: (c, 0)),
        ],
        out_specs=pl.BlockSpec((rows, 128), lambda c: (c, 0)),
        out_shape=jax.ShapeDtypeStruct(cat2.shape, jnp.int32),
    )(cat2, spx2)


def _sc_body(tbl_hbm, idx_hbm, out_hbm, idx_v, rows_v, out_v,
             semA, semB, semI0, semI1, semO0, semO1):
    wid = lax.axis_index("s") * NC + lax.axis_index("c")
    semG = (semA, semB)
    semI = (semI0, semI1)
    semO = (semO0, semO1)

    def idx_src(g):
        r0 = wid * (NCHUNK * NSTREAM) + g * NSTREAM
        return idx_hbm.at[pl.ds(r0, NSTREAM), :]

    def fire(buf):
        for r in range(NSTREAM):
            pltpu.async_copy(
                tbl_hbm.at[idx_v.at[buf, r]],
                rows_v.at[buf, pl.ds(r * 128, 128)],
                semG[buf],
            )

    def drain_rows(buf):
        pltpu.make_async_copy(
            tbl_hbm.at[pl.ds(0, LOOKUPS)], rows_v.at[buf], semG[buf]).wait()

    def drain_out(buf):
        pltpu.make_async_copy(
            out_v.at[buf], out_hbm.at[pl.ds(0, CHUNK)], semO[buf]).wait()

    def half(g, buf):
        drain_rows(buf)

        @pl.when(g + 2 < NCHUNK)
        def _():
            pltpu.async_copy(idx_src(g + 2), idx_v.at[buf], semI[buf])

        @pl.when(g >= 2)
        def _():
            drain_out(buf)

        def out_body(o, _):
            base = o * 16
            acc = rows_v[buf, base, :]
            for k in range(1, 16):
                acc = acc + rows_v[buf, base + k, :]
            out_v[buf, o, :] = acc
            return 0

        lax.fori_loop(0, CHUNK, out_body, 0)
        m0 = wid * M_PER_W + g * CHUNK
        pltpu.async_copy(out_v.at[buf], out_hbm.at[pl.ds(m0, CHUNK)],
                         semO[buf])

        @pl.when(g + 2 < NCHUNK)
        def _():
            pltpu.make_async_copy(idx_src(0), idx_v.at[buf], semI[buf]).wait()
            fire(buf)

    pltpu.sync_copy(idx_src(0), idx_v.at[0])
    fire(0)
    pltpu.sync_copy(idx_src(1), idx_v.at[1])
    fire(1)

    def body2(i, _):
        half(2 * i, 0)
        half(2 * i + 1, 1)
        return 0

    lax.fori_loop(0, NCHUNK // 2, body2, 0)
    drain_out(0)
    drain_out(1)


def _sc_gather(table, idx2):
    mesh = plsc.VectorSubcoreMesh(core_axis_name="c", subcore_axis_name="s")
    run = pl.kernel(
        _sc_body,
        out_type=jax.ShapeDtypeStruct((M, H), jnp.bfloat16),
        mesh=mesh,
        compiler_params=pltpu.CompilerParams(use_tc_tiling_on_sc=False),
        scratch_types=[
            pltpu.VMEM((2, NSTREAM, 128), jnp.int32),
            pltpu.VMEM((2, LOOKUPS, H), jnp.bfloat16),
            pltpu.VMEM((2, CHUNK, H), jnp.bfloat16),
            pltpu.SemaphoreType.DMA,
            pltpu.SemaphoreType.DMA,
            pltpu.SemaphoreType.DMA,
            pltpu.SemaphoreType.DMA,
            pltpu.SemaphoreType.DMA,
            pltpu.SemaphoreType.DMA,
        ],
    )
    return run(table, idx2)


def _assemble_body(ab_ref, e_ref, gt_ref, out_ref):
    et = e_ref[...].T.reshape(H, N, N).astype(jnp.float32)
    zr = jnp.zeros((H, 1, N), jnp.float32)
    tmp = jnp.concatenate([zr, et], axis=1)
    zc = jnp.zeros((H, N + 1, 1), jnp.float32)
    padded = jnp.concatenate([zc, tmp], axis=2)
    ii = lax.broadcasted_iota(jnp.int32, (H, N + 1, N + 1), 1)
    jj = lax.broadcasted_iota(jnp.int32, (H, N + 1, N + 1), 2)
    t = gt_ref[0, :].reshape(H, 1, 1)
    tfield = jnp.where((ii == 0) | (jj == 0), t, 0.0)
    out_ref[0] = 2.0 * ab_ref[0][None, :, :] + padded + tfield


def _assemble(attn_bias, eout, graph_token_w):
    return pl.pallas_call(
        _assemble_body,
        grid=(B,),
        in_specs=[
            pl.BlockSpec((1, N + 1, N + 1), lambda b: (b, 0, 0)),
            pl.BlockSpec((N * N, H), lambda b: (b, 0)),
            pl.BlockSpec((1, H), lambda b: (0, 0)),
        ],
        out_specs=pl.BlockSpec((1, H, N + 1, N + 1), lambda b: (b, 0, 0, 0)),
        out_shape=jax.ShapeDtypeStruct((B, H, N + 1, N + 1), jnp.float32),
    )(attn_bias, eout, graph_token_w)


def kernel(attn_bias, spatial_pos, x, edge_input, attn_edge_type,
           edge_encoder_w, spatial_pos_encoder_w, graph_token_w,
           edge_dis_encoder_w):
    spatial_pos = spatial_pos.astype(jnp.int32)
    edge_input = edge_input.astype(jnp.int32)

    e_pad = jnp.pad(edge_encoder_w, ((0, DSTRIDE - NE1), (0, 0)))
    wm = edge_dis_encoder_w.reshape(128, H, H)[:5]
    table = _build_table(e_pad, wm, spatial_pos_encoder_w)

    cat = jnp.concatenate(
        [edge_input.reshape(M, 15), spatial_pos.reshape(M, 1)], axis=1)
    spx = jnp.broadcast_to(spatial_pos.reshape(M, 1), (M, 16))
    idx2 = _build_idx(cat.reshape(IDX_ROWS, 128), spx.reshape(IDX_ROWS, 128))

    eout = _sc_gather(table, idx2)
    return _assemble(attn_bias, eout, graph_token_w)
